# baseline (speedup 1.0000x reference)
import jax
import jax.numpy as jnp
from jax import lax
from jax.experimental import pallas as pl
from jax.experimental.pallas import tpu as pltpu

B, R, N, D = 4, 4, 1024, 256
NTILE = 512
NT = N // NTILE
L = 2
KS = R * D
KX = KS + D
KTOT = KX + 128


def _body(adj_ref, x_ref, xown_ref, wcat_ref, out0_ref, out1_ref,
          acache_ref, x1_ref, rsmc_ref, den_ref,
          scat0_ref, scat1_ref, dacc_ref):
    bb = pl.program_id(0)
    n = pl.program_id(1)
    r = pl.program_id(2)

    @pl.when(bb < B)
    def _layer0():
        bn = bb * NT + n
        idx = bn * R + r
        adj_blk = adj_ref[0, 0]
        rowsum = jnp.sum(adj_blk, axis=1, keepdims=True)
        adj_bf = adj_blk.astype(jnp.bfloat16)
        acache_ref[idx] = adj_bf

        @pl.when(r == 0)
        def _():
            lane = lax.broadcasted_iota(jnp.int32, (NTILE, 128), 1)
            scat0_ref[:, KX:KTOT] = jnp.where(lane == R, 1.0, 0.0).astype(
                jnp.bfloat16)
            dacc_ref[...] = rowsum

        @pl.when(r > 0)
        def _():
            dacc_ref[...] += rowsum

        s = jnp.dot(adj_bf, x_ref[0], preferred_element_type=jnp.float32)
        sbf = s.astype(jnp.bfloat16)
        for k in range(R):
            @pl.when(r == k)
            def _():
                scat0_ref[:, k * D:(k + 1) * D] = sbf
                scat0_ref[:, KX + k:KX + k + 1] = rowsum.astype(jnp.bfloat16)

        @pl.when(r == R - 1)
        def _():
            den = dacc_ref[...] + 1.0
            den_ref[bn] = den
            scat0_ref[:, KS:KX] = xown_ref[0]
            rsmc_ref[bn] = scat0_ref[:, KX:KTOT]
            agg = jnp.dot(scat0_ref[...], wcat_ref[0, 0],
                          preferred_element_type=jnp.float32)
            out = jnp.maximum(agg / den, 0.0)
            out0_ref[0] = out
            x1_ref[bb, pl.ds(n * NTILE, NTILE)] = out.astype(jnp.bfloat16)

    @pl.when(bb >= 1)
    def _layer1():
        bp = bb - 1
        bn = bp * NT + n
        idx = bn * R + r
        s = jnp.dot(acache_ref[idx], x1_ref[bp],
                    preferred_element_type=jnp.float32)
        sbf = s.astype(jnp.bfloat16)
        for k in range(R):
            @pl.when(r == k)
            def _():
                scat1_ref[:, k * D:(k + 1) * D] = sbf

        @pl.when(r == R - 1)
        def _():
            scat1_ref[:, KS:KX] = x1_ref[bp, pl.ds(n * NTILE, NTILE)]
            scat1_ref[:, KX:KTOT] = rsmc_ref[bn]
            agg = jnp.dot(scat1_ref[...], wcat_ref[1, 0],
                          preferred_element_type=jnp.float32)
            out1_ref[0] = jnp.maximum(agg / den_ref[bn], 0.0)


@jax.jit
def kernel(nodes, adj, W0, b0, Wr, br):
    bf = jnp.bfloat16
    xbf = nodes.astype(bf)
    wr_t = Wr.transpose(0, 1, 3, 2).reshape(L, R * D, D)
    tail = jnp.zeros((L, 128, D), jnp.float32)
    tail = tail.at[:, :R, :].set(br).at[:, R, :].set(b0)
    wcat = jnp.concatenate(
        [wr_t, W0.transpose(0, 2, 1), tail], axis=1)[:, None].astype(bf)

    out0, out1 = pl.pallas_call(
        _body,
        grid=(B + 1, NT, R),
        in_specs=[
            pl.BlockSpec((1, 1, NTILE, N),
                         lambda bb, n, r: (jnp.minimum(bb, B - 1),
                                           jnp.where(bb < B, r, 0),
                                           jnp.where(bb < B, n, 0), 0)),
            pl.BlockSpec((1, N, D),
                         lambda bb, n, r: (jnp.minimum(bb, B - 1), 0, 0)),
            pl.BlockSpec((1, NTILE, D),
                         lambda bb, n, r: (jnp.minimum(bb, B - 1),
                                           jnp.where(bb < B, n, 0), 0)),
            pl.BlockSpec((L, 1, KTOT, D), lambda bb, n, r: (0, 0, 0, 0)),
        ],
        out_specs=[
            pl.BlockSpec((1, NTILE, D),
                         lambda bb, n, r: (jnp.minimum(bb, B - 1),
                                           jnp.where(bb < B, n, NT - 1), 0)),
            pl.BlockSpec((1, NTILE, D),
                         lambda bb, n, r: (jnp.maximum(bb - 1, 0),
                                           jnp.where(bb >= 1, n, 0), 0)),
        ],
        out_shape=[
            jax.ShapeDtypeStruct((B, N, D), jnp.float32),
            jax.ShapeDtypeStruct((B, N, D), jnp.float32),
        ],
        scratch_shapes=[
            pltpu.VMEM((B * NT * R, NTILE, N), jnp.bfloat16),
            pltpu.VMEM((B, N, D), jnp.bfloat16),
            pltpu.VMEM((B * NT, NTILE, 128), jnp.bfloat16),
            pltpu.VMEM((B * NT, NTILE, 1), jnp.float32),
            pltpu.VMEM((NTILE, KTOT), jnp.bfloat16),
            pltpu.VMEM((NTILE, KTOT), jnp.bfloat16),
            pltpu.VMEM((NTILE, 1), jnp.float32),
        ],
    )(adj, xbf, xbf, wcat)
    return (out0, out1)
